# initial kernel scaffold (unmeasured)
import jax
import jax.numpy as jnp
from jax import lax
from jax.experimental import pallas as pl
from jax.experimental.pallas import tpu as pltpu


def kernel(
    x,
):
    def body(*refs):
        pass

    out_shape = jax.ShapeDtypeStruct(..., jnp.float32)
    return pl.pallas_call(body, out_shape=out_shape)(...)


# baseline (device time: 298032 ns/iter reference)
import jax
import jax.numpy as jnp
from jax import lax
from jax.experimental import pallas as pl
from jax.experimental.pallas import tpu as pltpu

N_DEV = 4


def kernel(x):
    m, n = x.shape

    def body(x_ref, out_ref, comm_ref, send_sems, recv_sems):
        my = lax.axis_index("i")
        left = lax.rem(my - 1 + N_DEV, N_DEV)
        right = lax.rem(my + 1, N_DEV)

        barrier_sem = pltpu.get_barrier_semaphore()
        for nbr in (left, right):
            pl.semaphore_signal(
                barrier_sem, inc=1,
                device_id=(nbr,), device_id_type=pl.DeviceIdType.MESH,
            )
        pl.semaphore_wait(barrier_sem, 2)

        comm_ref[0] = x_ref[...].astype(jnp.bfloat16)
        out_ref[...] = comm_ref[0]

        for h in range(N_DEV - 1):
            rdma = pltpu.make_async_remote_copy(
                src_ref=comm_ref.at[h],
                dst_ref=comm_ref.at[h + 1],
                send_sem=send_sems.at[h],
                recv_sem=recv_sems.at[h],
                device_id=(right,),
                device_id_type=pl.DeviceIdType.MESH,
            )
            rdma.start()
            rdma.wait()
            out_ref[...] = out_ref[...] + comm_ref[h + 1]

    return pl.pallas_call(
        body,
        out_shape=jax.ShapeDtypeStruct((m, n), jnp.bfloat16),
        in_specs=[pl.BlockSpec(memory_space=pltpu.VMEM)],
        out_specs=pl.BlockSpec(memory_space=pltpu.VMEM),
        scratch_shapes=[
            pltpu.VMEM((N_DEV, m, n), jnp.bfloat16),
            pltpu.SemaphoreType.DMA((N_DEV - 1,)),
            pltpu.SemaphoreType.DMA((N_DEV - 1,)),
        ],
        compiler_params=pltpu.CompilerParams(
            collective_id=0,
            vmem_limit_bytes=100 * 1024 * 1024,
        ),
    )(x)


# device time: 94775 ns/iter; 3.1446x vs baseline; 3.1446x over previous
import jax
import jax.numpy as jnp
from jax import lax
from jax.experimental import pallas as pl
from jax.experimental.pallas import tpu as pltpu

N_DEV = 4


def kernel(x):
    m, n = x.shape
    mq = m // 2
    me = m // 4
    nh = n // 2

    def body(x_ref, out_ref, xb_ref, qrecv_ref, erecv_ref,
             send_sems, recv_sems):
        my = lax.axis_index("i")
        px = my // 2
        py = jnp.bitwise_and(jnp.bitwise_xor(my, px), 1)
        p_flip_y = jnp.bitwise_xor(my, 1)
        p_flip_x = 3 - my

        barrier_sem = pltpu.get_barrier_semaphore()
        for nbr in (p_flip_y, p_flip_x):
            pl.semaphore_signal(
                barrier_sem, inc=1,
                device_id=(nbr,), device_id_type=pl.DeviceIdType.MESH,
            )
        pl.semaphore_wait(barrier_sem, 2)

        xb_ref[...] = x_ref[...].astype(jnp.bfloat16)

        halves = []
        for h, (P, qi, ei) in enumerate([
            ((p_flip_y, p_flip_x, p_flip_x, p_flip_y), py, px),
            ((p_flip_x, p_flip_y, p_flip_y, p_flip_x), px, py),
        ]):
            halves.append((h, h * nh, P, qi, ei, qi * mq, qi * mq + ei * me))

        def sems(h, s):
            return send_sems.at[4 * h + s], recv_sems.at[4 * h + s]

        descs = []

        d1 = {}
        for h, c0, P, qi, ei, qb, eb in halves:
            ss, rs = sems(h, 0)
            d1[h] = pltpu.make_async_remote_copy(
                src_ref=xb_ref.at[pl.ds((1 - qi) * mq, mq), pl.ds(c0, nh)],
                dst_ref=qrecv_ref.at[h],
                send_sem=ss, recv_sem=rs,
                device_id=(P[0],), device_id_type=pl.DeviceIdType.MESH,
            )
            d1[h].start()
            descs.append(d1[h])
        for h, c0, P, qi, ei, qb, eb in halves:
            d1[h].wait_recv()
            qrecv_ref[h] = qrecv_ref[h] + xb_ref[pl.ds(qb, mq), pl.ds(c0, nh)]

        d2 = {}
        for h, c0, P, qi, ei, qb, eb in halves:
            ss, rs = sems(h, 1)
            d2[h] = pltpu.make_async_remote_copy(
                src_ref=qrecv_ref.at[h, pl.ds((1 - ei) * me, me), :],
                dst_ref=erecv_ref.at[h],
                send_sem=ss, recv_sem=rs,
                device_id=(P[1],), device_id_type=pl.DeviceIdType.MESH,
            )
            d2[h].start()
            descs.append(d2[h])
        for h, c0, P, qi, ei, qb, eb in halves:
            d2[h].wait_recv()
            out_ref[pl.ds(eb, me), pl.ds(c0, nh)] = (
                qrecv_ref[h, pl.ds(ei * me, me), :] + erecv_ref[h]
            )

        d3 = {}
        for h, c0, P, qi, ei, qb, eb in halves:
            ss, rs = sems(h, 2)
            d3[h] = pltpu.make_async_remote_copy(
                src_ref=out_ref.at[pl.ds(eb, me), pl.ds(c0, nh)],
                dst_ref=out_ref.at[pl.ds(eb, me), pl.ds(c0, nh)],
                send_sem=ss, recv_sem=rs,
                device_id=(P[2],), device_id_type=pl.DeviceIdType.MESH,
            )
            d3[h].start()
            descs.append(d3[h])
        for h, c0, P, qi, ei, qb, eb in halves:
            d3[h].wait_recv()

        d4 = {}
        for h, c0, P, qi, ei, qb, eb in halves:
            ss, rs = sems(h, 3)
            d4[h] = pltpu.make_async_remote_copy(
                src_ref=out_ref.at[pl.ds(qb, mq), pl.ds(c0, nh)],
                dst_ref=out_ref.at[pl.ds(qb, mq), pl.ds(c0, nh)],
                send_sem=ss, recv_sem=rs,
                device_id=(P[3],), device_id_type=pl.DeviceIdType.MESH,
            )
            d4[h].start()
            descs.append(d4[h])
        for h, c0, P, qi, ei, qb, eb in halves:
            d4[h].wait_recv()

        for d in descs:
            d.wait_send()

    return pl.pallas_call(
        body,
        out_shape=jax.ShapeDtypeStruct((m, n), jnp.bfloat16),
        in_specs=[pl.BlockSpec(memory_space=pltpu.VMEM)],
        out_specs=pl.BlockSpec(memory_space=pltpu.VMEM),
        scratch_shapes=[
            pltpu.VMEM((m, n), jnp.bfloat16),
            pltpu.VMEM((2, mq, nh), jnp.bfloat16),
            pltpu.VMEM((2, me, nh), jnp.bfloat16),
            pltpu.SemaphoreType.DMA((8,)),
            pltpu.SemaphoreType.DMA((8,)),
        ],
        compiler_params=pltpu.CompilerParams(
            collective_id=0,
            vmem_limit_bytes=100 * 1024 * 1024,
        ),
    )(x)


# device time: 90600 ns/iter; 3.2895x vs baseline; 1.0461x over previous
import jax
import jax.numpy as jnp
from jax import lax
from jax.experimental import pallas as pl
from jax.experimental.pallas import tpu as pltpu

N_DEV = 4
NSEM = 12


def kernel(x):
    m, n = x.shape
    mq = m // 2
    me = m // 4
    nh = n // 2
    f32 = jnp.float32
    bf16 = jnp.bfloat16

    def body(x_ref, out_ref, sbuf_ref, qrecv_ref, erecv_ref,
             send_sems, recv_sems):
        my = lax.axis_index("i")
        px = my // 2
        py = jnp.bitwise_and(jnp.bitwise_xor(my, px), 1)
        p_flip_y = jnp.bitwise_xor(my, 1)
        p_flip_x = 3 - my

        barrier_sem = pltpu.get_barrier_semaphore()
        for nbr in (p_flip_y, p_flip_x):
            pl.semaphore_signal(
                barrier_sem, inc=1,
                device_id=(nbr,), device_id_type=pl.DeviceIdType.MESH,
            )

        H = []
        for h, (P, qi, ei) in enumerate([
            ((p_flip_y, p_flip_x, p_flip_x, p_flip_y), py, px),
            ((p_flip_x, p_flip_y, p_flip_y, p_flip_x), px, py),
        ]):
            H.append((h, h * nh, P, qi, ei, qi * mq, qi * mq + ei * me))

        def xchg(src, dst, dev, h, s):
            return pltpu.make_async_remote_copy(
                src_ref=src, dst_ref=dst,
                send_sem=send_sems.at[6 * h + s],
                recv_sem=recv_sems.at[6 * h + s],
                device_id=(dev,), device_id_type=pl.DeviceIdType.MESH,
            )

        for h, c0, P, qi, ei, qb, eb in H:
            sbuf_ref[h, pl.ds((1 - ei) * me, me), :] = x_ref[
                pl.ds((1 - qi) * mq + (1 - ei) * me, me), pl.ds(c0, nh)
            ].astype(bf16)
        pl.semaphore_wait(barrier_sem, 2)

        descs = []
        d1a, d1b, d2, d3, d4a, d4b = {}, {}, {}, {}, {}, {}

        for h, c0, P, qi, ei, qb, eb in H:
            d1a[h] = xchg(
                sbuf_ref.at[h, pl.ds((1 - ei) * me, me), :],
                qrecv_ref.at[h, pl.ds((1 - ei) * me, me), :],
                P[0], h, 0,
            )
            d1a[h].start()
            descs.append(d1a[h])
        for h, c0, P, qi, ei, qb, eb in H:
            sbuf_ref[h, pl.ds(ei * me, me), :] = x_ref[
                pl.ds((1 - qi) * mq + ei * me, me), pl.ds(c0, nh)
            ].astype(bf16)
            d1b[h] = xchg(
                sbuf_ref.at[h, pl.ds(ei * me, me), :],
                qrecv_ref.at[h, pl.ds(ei * me, me), :],
                P[0], h, 1,
            )
            d1b[h].start()
            descs.append(d1b[h])

        for h, c0, P, qi, ei, qb, eb in H:
            d1a[h].wait_recv()
            fwd = pl.ds((1 - ei) * me, me)
            qrecv_ref[h, fwd, :] = (
                x_ref[pl.ds(qb + (1 - ei) * me, me), pl.ds(c0, nh)]
                + qrecv_ref[h, fwd, :].astype(f32)
            ).astype(bf16)
            d2[h] = xchg(
                qrecv_ref.at[h, fwd, :], erecv_ref.at[h], P[1], h, 2,
            )
            d2[h].start()
            descs.append(d2[h])

        for h, c0, P, qi, ei, qb, eb in H:
            d1b[h].wait_recv()
            d2[h].wait_recv()
            out_ref[pl.ds(eb, me), pl.ds(c0, nh)] = (
                x_ref[pl.ds(eb, me), pl.ds(c0, nh)]
                + qrecv_ref[h, pl.ds(ei * me, me), :].astype(f32)
                + erecv_ref[h].astype(f32)
            ).astype(bf16)

        for h, c0, P, qi, ei, qb, eb in H:
            d3[h] = xchg(
                out_ref.at[pl.ds(eb, me), pl.ds(c0, nh)],
                out_ref.at[pl.ds(eb, me), pl.ds(c0, nh)],
                P[2], h, 3,
            )
            d3[h].start()
            descs.append(d3[h])
        for h, c0, P, qi, ei, qb, eb in H:
            d4a[h] = xchg(
                out_ref.at[pl.ds(eb, me), pl.ds(c0, nh)],
                out_ref.at[pl.ds(eb, me), pl.ds(c0, nh)],
                P[3], h, 4,
            )
            d4a[h].start()
            descs.append(d4a[h])

        for h, c0, P, qi, ei, qb, eb in H:
            d3[h].wait_recv()
            oth = pl.ds(qb + (1 - ei) * me, me)
            d4b[h] = xchg(
                out_ref.at[oth, pl.ds(c0, nh)],
                out_ref.at[oth, pl.ds(c0, nh)],
                P[3], h, 5,
            )
            d4b[h].start()
            descs.append(d4b[h])

        for h, c0, P, qi, ei, qb, eb in H:
            d4a[h].wait_recv()
            d4b[h].wait_recv()
        for d in descs:
            d.wait_send()

    return pl.pallas_call(
        body,
        out_shape=jax.ShapeDtypeStruct((m, n), bf16),
        in_specs=[pl.BlockSpec(memory_space=pltpu.VMEM)],
        out_specs=pl.BlockSpec(memory_space=pltpu.VMEM),
        scratch_shapes=[
            pltpu.VMEM((2, mq, nh), bf16),
            pltpu.VMEM((2, mq, nh), bf16),
            pltpu.VMEM((2, me, nh), bf16),
            pltpu.SemaphoreType.DMA((NSEM,)),
            pltpu.SemaphoreType.DMA((NSEM,)),
        ],
        compiler_params=pltpu.CompilerParams(
            collective_id=0,
            vmem_limit_bytes=100 * 1024 * 1024,
        ),
    )(x)


# device time: 90512 ns/iter; 3.2927x vs baseline; 1.0010x over previous
import jax
import jax.numpy as jnp
from jax import lax
from jax.experimental import pallas as pl
from jax.experimental.pallas import tpu as pltpu

N_DEV = 4
NSEM = 12


def kernel(x):
    m, n = x.shape
    hm = m // 2
    mq = m // 4
    me = m // 8
    f32 = jnp.float32
    bf16 = jnp.bfloat16

    def body(x_ref, out_ref, sbuf_ref, qrecv_ref, erecv_ref,
             send_sems, recv_sems):
        my = lax.axis_index("i")
        px = my // 2
        py = jnp.bitwise_and(jnp.bitwise_xor(my, px), 1)
        p_flip_y = jnp.bitwise_xor(my, 1)
        p_flip_x = 3 - my

        barrier_sem = pltpu.get_barrier_semaphore()
        for nbr in (p_flip_y, p_flip_x):
            pl.semaphore_signal(
                barrier_sem, inc=1,
                device_id=(nbr,), device_id_type=pl.DeviceIdType.MESH,
            )

        H = []
        for h, (P, qi, ei) in enumerate([
            ((p_flip_y, p_flip_x, p_flip_x, p_flip_y), py, px),
            ((p_flip_x, p_flip_y, p_flip_y, p_flip_x), px, py),
        ]):
            hb = h * hm
            H.append((h, hb, P, qi, ei, hb + qi * mq, hb + qi * mq + ei * me))

        def xchg(src, dst, dev, h, s):
            return pltpu.make_async_remote_copy(
                src_ref=src, dst_ref=dst,
                send_sem=send_sems.at[6 * h + s],
                recv_sem=recv_sems.at[6 * h + s],
                device_id=(dev,), device_id_type=pl.DeviceIdType.MESH,
            )

        for h, hb, P, qi, ei, qb, eb in H:
            sbuf_ref[h, pl.ds((1 - ei) * me, me), :] = x_ref[
                pl.ds(hb + (1 - qi) * mq + (1 - ei) * me, me), :
            ].astype(bf16)
        pl.semaphore_wait(barrier_sem, 2)

        descs = []
        d1a, d1b, d2, d3, d4a, d4b = {}, {}, {}, {}, {}, {}

        for h, hb, P, qi, ei, qb, eb in H:
            d1a[h] = xchg(
                sbuf_ref.at[h, pl.ds((1 - ei) * me, me), :],
                qrecv_ref.at[h, pl.ds((1 - ei) * me, me), :],
                P[0], h, 0,
            )
            d1a[h].start()
            descs.append(d1a[h])
        for h, hb, P, qi, ei, qb, eb in H:
            sbuf_ref[h, pl.ds(ei * me, me), :] = x_ref[
                pl.ds(hb + (1 - qi) * mq + ei * me, me), :
            ].astype(bf16)
            d1b[h] = xchg(
                sbuf_ref.at[h, pl.ds(ei * me, me), :],
                qrecv_ref.at[h, pl.ds(ei * me, me), :],
                P[0], h, 1,
            )
            d1b[h].start()
            descs.append(d1b[h])

        for h, hb, P, qi, ei, qb, eb in H:
            d1a[h].wait_recv()
            fwd = pl.ds((1 - ei) * me, me)
            qrecv_ref[h, fwd, :] = (
                x_ref[pl.ds(qb + (1 - ei) * me, me), :]
                + qrecv_ref[h, fwd, :].astype(f32)
            ).astype(bf16)
            d2[h] = xchg(
                qrecv_ref.at[h, fwd, :], erecv_ref.at[h], P[1], h, 2,
            )
            d2[h].start()
            descs.append(d2[h])

        for h, hb, P, qi, ei, qb, eb in H:
            d1b[h].wait_recv()
            d2[h].wait_recv()
            out_ref[pl.ds(eb, me), :] = (
                x_ref[pl.ds(eb, me), :]
                + qrecv_ref[h, pl.ds(ei * me, me), :].astype(f32)
                + erecv_ref[h].astype(f32)
            ).astype(bf16)

        for h, hb, P, qi, ei, qb, eb in H:
            d3[h] = xchg(
                out_ref.at[pl.ds(eb, me), :],
                out_ref.at[pl.ds(eb, me), :],
                P[2], h, 3,
            )
            d3[h].start()
            descs.append(d3[h])
        for h, hb, P, qi, ei, qb, eb in H:
            d4a[h] = xchg(
                out_ref.at[pl.ds(eb, me), :],
                out_ref.at[pl.ds(eb, me), :],
                P[3], h, 4,
            )
            d4a[h].start()
            descs.append(d4a[h])

        for h, hb, P, qi, ei, qb, eb in H:
            d3[h].wait_recv()
            oth = pl.ds(qb + (1 - ei) * me, me)
            d4b[h] = xchg(
                out_ref.at[oth, :],
                out_ref.at[oth, :],
                P[3], h, 5,
            )
            d4b[h].start()
            descs.append(d4b[h])

        for h, hb, P, qi, ei, qb, eb in H:
            d4a[h].wait_recv()
            d4b[h].wait_recv()
        for d in descs:
            d.wait_send()

    return pl.pallas_call(
        body,
        out_shape=jax.ShapeDtypeStruct((m, n), bf16),
        in_specs=[pl.BlockSpec(memory_space=pltpu.VMEM)],
        out_specs=pl.BlockSpec(memory_space=pltpu.VMEM),
        scratch_shapes=[
            pltpu.VMEM((2, mq, n), bf16),
            pltpu.VMEM((2, mq, n), bf16),
            pltpu.VMEM((2, me, n), bf16),
            pltpu.SemaphoreType.DMA((NSEM,)),
            pltpu.SemaphoreType.DMA((NSEM,)),
        ],
        compiler_params=pltpu.CompilerParams(
            collective_id=0,
            vmem_limit_bytes=100 * 1024 * 1024,
        ),
    )(x)


# device time: 89749 ns/iter; 3.3207x vs baseline; 1.0085x over previous
import jax
import jax.numpy as jnp
from jax import lax
from jax.experimental import pallas as pl
from jax.experimental.pallas import tpu as pltpu

N_DEV = 4
NSEM = 12


def kernel(x):
    m, n = x.shape
    hm = m // 2
    mq = m // 4
    me = m // 8
    f32 = jnp.float32
    bf16 = jnp.bfloat16

    def body(x_ref, out_ref, xv_ref, sbuf_ref, qrecv_ref, erecv_ref,
             send_sems, recv_sems, fetch_sems):
        my = lax.axis_index("i")
        px = my // 2
        py = jnp.bitwise_and(jnp.bitwise_xor(my, px), 1)
        p_flip_y = jnp.bitwise_xor(my, 1)
        p_flip_x = 3 - my

        H = []
        for h, (P, qi, ei) in enumerate([
            ((p_flip_y, p_flip_x, p_flip_x, p_flip_y), py, px),
            ((p_flip_x, p_flip_y, p_flip_y, p_flip_x), px, py),
        ]):
            hb = h * hm
            H.append((h, hb, P, qi, ei, hb + qi * mq, hb + qi * mq + ei * me))

        fetch = {}
        for h, hb, P, qi, ei, qb, eb in H:
            sq = hb + (1 - qi) * mq
            for k, r in enumerate(
                (sq + (1 - ei) * me, sq + ei * me, qb + (1 - ei) * me, eb)
            ):
                cp = pltpu.make_async_copy(
                    x_ref.at[pl.ds(r, me), :],
                    xv_ref.at[pl.ds(r, me), :],
                    fetch_sems.at[4 * h + k],
                )
                cp.start()
                fetch[(h, k)] = (cp, r)

        barrier_sem = pltpu.get_barrier_semaphore()
        for nbr in (p_flip_y, p_flip_x):
            pl.semaphore_signal(
                barrier_sem, inc=1,
                device_id=(nbr,), device_id_type=pl.DeviceIdType.MESH,
            )

        def xchg(src, dst, dev, h, s):
            return pltpu.make_async_remote_copy(
                src_ref=src, dst_ref=dst,
                send_sem=send_sems.at[6 * h + s],
                recv_sem=recv_sems.at[6 * h + s],
                device_id=(dev,), device_id_type=pl.DeviceIdType.MESH,
            )

        for h, hb, P, qi, ei, qb, eb in H:
            cp, r = fetch[(h, 0)]
            cp.wait()
            sbuf_ref[h, pl.ds((1 - ei) * me, me), :] = xv_ref[
                pl.ds(r, me), :
            ].astype(bf16)
        pl.semaphore_wait(barrier_sem, 2)

        descs = []
        d1a, d1b, d2, d3, d4a, d4b = {}, {}, {}, {}, {}, {}

        for h, hb, P, qi, ei, qb, eb in H:
            d1a[h] = xchg(
                sbuf_ref.at[h, pl.ds((1 - ei) * me, me), :],
                qrecv_ref.at[h, pl.ds((1 - ei) * me, me), :],
                P[0], h, 0,
            )
            d1a[h].start()
            descs.append(d1a[h])
        for h, hb, P, qi, ei, qb, eb in H:
            cp, r = fetch[(h, 1)]
            cp.wait()
            sbuf_ref[h, pl.ds(ei * me, me), :] = xv_ref[
                pl.ds(r, me), :
            ].astype(bf16)
            d1b[h] = xchg(
                sbuf_ref.at[h, pl.ds(ei * me, me), :],
                qrecv_ref.at[h, pl.ds(ei * me, me), :],
                P[0], h, 1,
            )
            d1b[h].start()
            descs.append(d1b[h])

        for h, hb, P, qi, ei, qb, eb in H:
            fetch[(h, 2)][0].wait()
            d1a[h].wait_recv()
            fwd = pl.ds((1 - ei) * me, me)
            qrecv_ref[h, fwd, :] = (
                xv_ref[pl.ds(qb + (1 - ei) * me, me), :]
                + qrecv_ref[h, fwd, :].astype(f32)
            ).astype(bf16)
            d2[h] = xchg(
                qrecv_ref.at[h, fwd, :], erecv_ref.at[h], P[1], h, 2,
            )
            d2[h].start()
            descs.append(d2[h])

        for h, hb, P, qi, ei, qb, eb in H:
            fetch[(h, 3)][0].wait()
            d1b[h].wait_recv()
            out_ref[pl.ds(eb, me), :] = (
                xv_ref[pl.ds(eb, me), :]
                + qrecv_ref[h, pl.ds(ei * me, me), :].astype(f32)
            ).astype(bf16)

        for h, hb, P, qi, ei, qb, eb in H:
            d2[h].wait_recv()
            out_ref[pl.ds(eb, me), :] = (
                out_ref[pl.ds(eb, me), :] + erecv_ref[h]
            )
            d3[h] = xchg(
                out_ref.at[pl.ds(eb, me), :],
                out_ref.at[pl.ds(eb, me), :],
                P[2], h, 3,
            )
            d3[h].start()
            descs.append(d3[h])
            d4a[h] = xchg(
                out_ref.at[pl.ds(eb, me), :],
                out_ref.at[pl.ds(eb, me), :],
                P[3], h, 4,
            )
            d4a[h].start()
            descs.append(d4a[h])

        for h, hb, P, qi, ei, qb, eb in H:
            d3[h].wait_recv()
            oth = pl.ds(qb + (1 - ei) * me, me)
            d4b[h] = xchg(
                out_ref.at[oth, :],
                out_ref.at[oth, :],
                P[3], h, 5,
            )
            d4b[h].start()
            descs.append(d4b[h])

        for h, hb, P, qi, ei, qb, eb in H:
            d4a[h].wait_recv()
            d4b[h].wait_recv()
        for d in descs:
            d.wait_send()

    return pl.pallas_call(
        body,
        out_shape=jax.ShapeDtypeStruct((m, n), bf16),
        in_specs=[pl.BlockSpec(memory_space=pl.ANY)],
        out_specs=pl.BlockSpec(memory_space=pltpu.VMEM),
        scratch_shapes=[
            pltpu.VMEM((m, n), jnp.float32),
            pltpu.VMEM((2, mq, n), bf16),
            pltpu.VMEM((2, mq, n), bf16),
            pltpu.VMEM((2, me, n), bf16),
            pltpu.SemaphoreType.DMA((NSEM,)),
            pltpu.SemaphoreType.DMA((NSEM,)),
            pltpu.SemaphoreType.DMA((8,)),
        ],
        compiler_params=pltpu.CompilerParams(
            collective_id=0,
            vmem_limit_bytes=100 * 1024 * 1024,
        ),
    )(x)


# device time: 87493 ns/iter; 3.4064x vs baseline; 1.0258x over previous
import jax
import jax.numpy as jnp
from jax import lax
from jax.experimental import pallas as pl
from jax.experimental.pallas import tpu as pltpu

N_DEV = 4
NSEM = 12


def kernel(x):
    m, n = x.shape
    hm = m // 2
    mq = m // 4
    me = m // 8
    f32 = jnp.float32
    bf16 = jnp.bfloat16

    def body(x_ref, out_ref, xv_ref, sbuf_ref, qrecv_ref, erecv_ref, ov_ref,
             gbuf_ref, send_sems, recv_sems, fetch_sems, wb_sems):
        my = lax.axis_index("i")
        px = my // 2
        py = jnp.bitwise_and(jnp.bitwise_xor(my, px), 1)
        p_flip_y = jnp.bitwise_xor(my, 1)
        p_flip_x = 3 - my

        H = []
        for h, (P, qi, ei) in enumerate([
            ((p_flip_y, p_flip_x, p_flip_x, p_flip_y), py, px),
            ((p_flip_x, p_flip_y, p_flip_y, p_flip_x), px, py),
        ]):
            hb = h * hm
            H.append((h, hb, P, qi, ei, hb + qi * mq, hb + qi * mq + ei * me))

        fetch = {}
        for h, hb, P, qi, ei, qb, eb in H:
            sq = hb + (1 - qi) * mq
            for k, r in enumerate(
                (sq + (1 - ei) * me, sq + ei * me, qb + (1 - ei) * me, eb)
            ):
                cp = pltpu.make_async_copy(
                    x_ref.at[pl.ds(r, me), :],
                    xv_ref.at[pl.ds(r, me), :],
                    fetch_sems.at[4 * h + k],
                )
                cp.start()
                fetch[(h, k)] = (cp, r)

        barrier_sem = pltpu.get_barrier_semaphore()
        for nbr in (p_flip_y, p_flip_x):
            pl.semaphore_signal(
                barrier_sem, inc=1,
                device_id=(nbr,), device_id_type=pl.DeviceIdType.MESH,
            )

        def xchg(src, dst, dev, h, s):
            return pltpu.make_async_remote_copy(
                src_ref=src, dst_ref=dst,
                send_sem=send_sems.at[6 * h + s],
                recv_sem=recv_sems.at[6 * h + s],
                device_id=(dev,), device_id_type=pl.DeviceIdType.MESH,
            )

        for h, hb, P, qi, ei, qb, eb in H:
            cp, r = fetch[(h, 0)]
            cp.wait()
            sbuf_ref[h, pl.ds((1 - ei) * me, me), :] = xv_ref[
                pl.ds(r, me), :
            ].astype(bf16)
        pl.semaphore_wait(barrier_sem, 2)

        descs = []
        d1a, d1b, d2, d3, d4a, d4b = {}, {}, {}, {}, {}, {}

        for h, hb, P, qi, ei, qb, eb in H:
            d1a[h] = xchg(
                sbuf_ref.at[h, pl.ds((1 - ei) * me, me), :],
                qrecv_ref.at[h, pl.ds((1 - ei) * me, me), :],
                P[0], h, 0,
            )
            d1a[h].start()
            descs.append(d1a[h])
        for h, hb, P, qi, ei, qb, eb in H:
            cp, r = fetch[(h, 1)]
            cp.wait()
            sbuf_ref[h, pl.ds(ei * me, me), :] = xv_ref[
                pl.ds(r, me), :
            ].astype(bf16)
            d1b[h] = xchg(
                sbuf_ref.at[h, pl.ds(ei * me, me), :],
                qrecv_ref.at[h, pl.ds(ei * me, me), :],
                P[0], h, 1,
            )
            d1b[h].start()
            descs.append(d1b[h])

        for h, hb, P, qi, ei, qb, eb in H:
            fetch[(h, 2)][0].wait()
            d1a[h].wait_recv()
            fwd = pl.ds((1 - ei) * me, me)
            qrecv_ref[h, fwd, :] = (
                xv_ref[pl.ds(qb + (1 - ei) * me, me), :]
                + qrecv_ref[h, fwd, :].astype(f32)
            ).astype(bf16)
            d2[h] = xchg(
                qrecv_ref.at[h, fwd, :], erecv_ref.at[h], P[1], h, 2,
            )
            d2[h].start()
            descs.append(d2[h])

        for h, hb, P, qi, ei, qb, eb in H:
            fetch[(h, 3)][0].wait()
            d1b[h].wait_recv()
            ov_ref[h] = (
                xv_ref[pl.ds(eb, me), :]
                + qrecv_ref[h, pl.ds(ei * me, me), :].astype(f32)
            ).astype(bf16)

        wb = {}
        for h, hb, P, qi, ei, qb, eb in H:
            d2[h].wait_recv()
            ov_ref[h] = ov_ref[h] + erecv_ref[h]
            own = pl.ds(qi * mq + ei * me, me)
            d3[h] = xchg(ov_ref.at[h], gbuf_ref.at[h, own, :], P[2], h, 3)
            d3[h].start()
            descs.append(d3[h])
            d4a[h] = xchg(ov_ref.at[h], gbuf_ref.at[h, own, :], P[3], h, 4)
            d4a[h].start()
            descs.append(d4a[h])
            wb[(h, 0)] = pltpu.make_async_copy(
                ov_ref.at[h], out_ref.at[pl.ds(eb, me), :], wb_sems.at[4 * h],
            )
            wb[(h, 0)].start()

        for h, hb, P, qi, ei, qb, eb in H:
            d3[h].wait_recv()
            oth = pl.ds(qi * mq + (1 - ei) * me, me)
            d4b[h] = xchg(
                gbuf_ref.at[h, oth, :], gbuf_ref.at[h, oth, :], P[3], h, 5,
            )
            d4b[h].start()
            descs.append(d4b[h])
            wb[(h, 1)] = pltpu.make_async_copy(
                gbuf_ref.at[h, oth, :],
                out_ref.at[pl.ds(qb + (1 - ei) * me, me), :],
                wb_sems.at[4 * h + 1],
            )
            wb[(h, 1)].start()

        for h, hb, P, qi, ei, qb, eb in H:
            oq = (1 - qi) * mq
            d4a[h].wait_recv()
            wb[(h, 2)] = pltpu.make_async_copy(
                gbuf_ref.at[h, pl.ds(oq + ei * me, me), :],
                out_ref.at[pl.ds(hb + oq + ei * me, me), :],
                wb_sems.at[4 * h + 2],
            )
            wb[(h, 2)].start()
            d4b[h].wait_recv()
            wb[(h, 3)] = pltpu.make_async_copy(
                gbuf_ref.at[h, pl.ds(oq + (1 - ei) * me, me), :],
                out_ref.at[pl.ds(hb + oq + (1 - ei) * me, me), :],
                wb_sems.at[4 * h + 3],
            )
            wb[(h, 3)].start()

        for k in wb:
            wb[k].wait()
        for d in descs:
            d.wait_send()

    return pl.pallas_call(
        body,
        out_shape=jax.ShapeDtypeStruct((m, n), bf16),
        in_specs=[pl.BlockSpec(memory_space=pl.ANY)],
        out_specs=pl.BlockSpec(memory_space=pl.ANY),
        scratch_shapes=[
            pltpu.VMEM((m, n), jnp.float32),
            pltpu.VMEM((2, mq, n), bf16),
            pltpu.VMEM((2, mq, n), bf16),
            pltpu.VMEM((2, me, n), bf16),
            pltpu.VMEM((2, me, n), bf16),
            pltpu.VMEM((2, hm, n), bf16),
            pltpu.SemaphoreType.DMA((NSEM,)),
            pltpu.SemaphoreType.DMA((NSEM,)),
            pltpu.SemaphoreType.DMA((8,)),
            pltpu.SemaphoreType.DMA((8,)),
        ],
        compiler_params=pltpu.CompilerParams(
            collective_id=0,
            vmem_limit_bytes=100 * 1024 * 1024,
        ),
    )(x)


# device time: 84688 ns/iter; 3.5192x vs baseline; 1.0331x over previous
import jax
import jax.numpy as jnp
from jax import lax
from jax.experimental import pallas as pl
from jax.experimental.pallas import tpu as pltpu

N_DEV = 4
NSEM = 20
NWB = 14


def kernel(x):
    m, n = x.shape
    hm = m // 2
    mq = m // 4
    me = m // 8
    qme = m // 16
    f32 = jnp.float32
    bf16 = jnp.bfloat16

    def body(x_ref, out_ref, xv_ref, sbuf_ref, qrecv_ref, erecv_ref, ov_ref,
             gbuf_ref, send_sems, recv_sems, fetch_sems, wb_sems):
        my = lax.axis_index("i")
        px = my // 2
        py = jnp.bitwise_and(jnp.bitwise_xor(my, px), 1)
        p_flip_y = jnp.bitwise_xor(my, 1)
        p_flip_x = 3 - my

        H = []
        for h, (P, qi, ei) in enumerate([
            ((p_flip_y, p_flip_x, p_flip_x, p_flip_y), py, px),
            ((p_flip_x, p_flip_y, p_flip_y, p_flip_x), px, py),
        ]):
            hb = h * hm
            H.append((h, hb, P, qi, ei, hb + qi * mq, hb + qi * mq + ei * me))

        fetch = {}
        for h, hb, P, qi, ei, qb, eb in H:
            sq = hb + (1 - qi) * mq
            for k, r in enumerate(
                (sq + (1 - ei) * me, sq + ei * me, qb + (1 - ei) * me, eb)
            ):
                cp = pltpu.make_async_copy(
                    x_ref.at[pl.ds(r, me), :],
                    xv_ref.at[pl.ds(r, me), :],
                    fetch_sems.at[4 * h + k],
                )
                cp.start()
                fetch[(h, k)] = (cp, r)

        barrier_sem = pltpu.get_barrier_semaphore()
        for nbr in (p_flip_y, p_flip_x):
            pl.semaphore_signal(
                barrier_sem, inc=1,
                device_id=(nbr,), device_id_type=pl.DeviceIdType.MESH,
            )

        def xchg(src, dst, dev, h, s):
            return pltpu.make_async_remote_copy(
                src_ref=src, dst_ref=dst,
                send_sem=send_sems.at[10 * h + s],
                recv_sem=recv_sems.at[10 * h + s],
                device_id=(dev,), device_id_type=pl.DeviceIdType.MESH,
            )

        for h, hb, P, qi, ei, qb, eb in H:
            cp, r = fetch[(h, 0)]
            cp.wait()
            sbuf_ref[h, pl.ds((1 - ei) * me, me), :] = xv_ref[
                pl.ds(r, me), :
            ].astype(bf16)
        pl.semaphore_wait(barrier_sem, 2)

        descs = []
        d1a, d1b, d2, d3, d4a, d4b = {}, {}, {}, {}, {}, {}
        wb = {}

        for h, hb, P, qi, ei, qb, eb in H:
            d1a[h] = xchg(
                sbuf_ref.at[h, pl.ds((1 - ei) * me, me), :],
                qrecv_ref.at[h, pl.ds((1 - ei) * me, me), :],
                P[0], h, 0,
            )
            d1a[h].start()
            descs.append(d1a[h])
        for h, hb, P, qi, ei, qb, eb in H:
            cp, r = fetch[(h, 1)]
            cp.wait()
            sbuf_ref[h, pl.ds(ei * me, me), :] = xv_ref[
                pl.ds(r, me), :
            ].astype(bf16)
            d1b[h] = xchg(
                sbuf_ref.at[h, pl.ds(ei * me, me), :],
                qrecv_ref.at[h, pl.ds(ei * me, me), :],
                P[0], h, 1,
            )
            d1b[h].start()
            descs.append(d1b[h])

        for h, hb, P, qi, ei, qb, eb in H:
            fetch[(h, 2)][0].wait()
            d1a[h].wait_recv()
            fwd = pl.ds((1 - ei) * me, me)
            qrecv_ref[h, fwd, :] = (
                xv_ref[pl.ds(qb + (1 - ei) * me, me), :]
                + qrecv_ref[h, fwd, :].astype(f32)
            ).astype(bf16)
            for c in (0, 1):
                fc = pl.ds((1 - ei) * me + c * qme, qme)
                d2[(h, c)] = xchg(
                    qrecv_ref.at[h, fc, :],
                    erecv_ref.at[h, pl.ds(c * qme, qme), :],
                    P[1], h, 2 + c,
                )
                d2[(h, c)].start()
                descs.append(d2[(h, c)])

        for h, hb, P, qi, ei, qb, eb in H:
            fetch[(h, 3)][0].wait()
            d1b[h].wait_recv()
            ov_ref[h] = (
                xv_ref[pl.ds(eb, me), :]
                + qrecv_ref[h, pl.ds(ei * me, me), :].astype(f32)
            ).astype(bf16)

        for h, hb, P, qi, ei, qb, eb in H:
            own0 = qi * mq + ei * me
            for c in (0, 1):
                cc = pl.ds(c * qme, qme)
                d2[(h, c)].wait_recv()
                ov_ref[h, cc, :] = ov_ref[h, cc, :] + erecv_ref[h, cc, :]
                ownc = pl.ds(own0 + c * qme, qme)
                d3[(h, c)] = xchg(
                    ov_ref.at[h, cc, :], gbuf_ref.at[h, ownc, :],
                    P[2], h, 4 + c,
                )
                d3[(h, c)].start()
                descs.append(d3[(h, c)])
                d4a[(h, c)] = xchg(
                    ov_ref.at[h, cc, :], gbuf_ref.at[h, ownc, :],
                    P[3], h, 6 + c,
                )
                d4a[(h, c)].start()
                descs.append(d4a[(h, c)])
            wb[(h, 0)] = pltpu.make_async_copy(
                ov_ref.at[h], out_ref.at[pl.ds(eb, me), :], wb_sems.at[7 * h],
            )
            wb[(h, 0)].start()

        for h, hb, P, qi, ei, qb, eb in H:
            oth0 = qi * mq + (1 - ei) * me
            for c in (0, 1):
                d3[(h, c)].wait_recv()
                oc = pl.ds(oth0 + c * qme, qme)
                d4b[(h, c)] = xchg(
                    gbuf_ref.at[h, oc, :], gbuf_ref.at[h, oc, :],
                    P[3], h, 8 + c,
                )
                d4b[(h, c)].start()
                descs.append(d4b[(h, c)])
                wb[(h, 1, c)] = pltpu.make_async_copy(
                    gbuf_ref.at[h, oc, :],
                    out_ref.at[pl.ds(hb + oth0 + c * qme, qme), :],
                    wb_sems.at[7 * h + 1 + c],
                )
                wb[(h, 1, c)].start()

        for h, hb, P, qi, ei, qb, eb in H:
            oq = (1 - qi) * mq
            for c in (0, 1):
                d4a[(h, c)].wait_recv()
                r = oq + ei * me + c * qme
                wb[(h, 2, c)] = pltpu.make_async_copy(
                    gbuf_ref.at[h, pl.ds(r, qme), :],
                    out_ref.at[pl.ds(hb + r, qme), :],
                    wb_sems.at[7 * h + 3 + c],
                )
                wb[(h, 2, c)].start()
            for c in (0, 1):
                d4b[(h, c)].wait_recv()
                r = oq + (1 - ei) * me + c * qme
                wb[(h, 3, c)] = pltpu.make_async_copy(
                    gbuf_ref.at[h, pl.ds(r, qme), :],
                    out_ref.at[pl.ds(hb + r, qme), :],
                    wb_sems.at[7 * h + 5 + c],
                )
                wb[(h, 3, c)].start()

        for k in wb:
            wb[k].wait()
        for d in descs:
            d.wait_send()

    return pl.pallas_call(
        body,
        out_shape=jax.ShapeDtypeStruct((m, n), bf16),
        in_specs=[pl.BlockSpec(memory_space=pl.ANY)],
        out_specs=pl.BlockSpec(memory_space=pl.ANY),
        scratch_shapes=[
            pltpu.VMEM((m, n), jnp.float32),
            pltpu.VMEM((2, mq, n), bf16),
            pltpu.VMEM((2, mq, n), bf16),
            pltpu.VMEM((2, me, n), bf16),
            pltpu.VMEM((2, me, n), bf16),
            pltpu.VMEM((2, hm, n), bf16),
            pltpu.SemaphoreType.DMA((NSEM,)),
            pltpu.SemaphoreType.DMA((NSEM,)),
            pltpu.SemaphoreType.DMA((8,)),
            pltpu.SemaphoreType.DMA((NWB,)),
        ],
        compiler_params=pltpu.CompilerParams(
            collective_id=0,
            vmem_limit_bytes=100 * 1024 * 1024,
        ),
    )(x)
